# Initial kernel scaffold; baseline (speedup 1.0000x reference)
import jax
import jax.numpy as jnp
from jax.experimental import pallas as pl


def kernel(x, edge_index, W, b):
    raise NotImplementedError("write your pallas kernel here")


# speedup vs baseline: 3.5363x; 3.5363x over previous
import functools

import jax
import jax.numpy as jnp
from jax import lax
from jax.experimental import pallas as pl
from jax.experimental.pallas import tpu as pltpu
from jax.experimental.pallas import tpu_sc as plsc

N = 10000
E = 320000
D = 128
K = 10
ALPHA = 0.1

NS = 16
EPAD = 2560 * 128
EROWS = EPAD // 128
EROWS_PT = EROWS // NS
JBLK = 8
NBLK = EROWS_PT // JBLK
NPAD = 10240
ROWS_PT = NPAD // NS
RCH = 32
NCH = ROWS_PT // RCH
ZR = 8
DEGPAD = NPAD


def _deg_body(col2d_hbm, deg_hbm, deg_sh, idx_v, ones_v, zero_v, sem):
    del sem
    sid = lax.axis_index("s")

    ones16 = jnp.ones((16,), jnp.float32)
    zeros16 = jnp.zeros((16,), jnp.float32)

    def fill(i, _):
        ones_v[pl.ds(i * 16, 16)] = ones16
        return 0

    lax.fori_loop(0, 128 // 16, fill, 0)

    def fillz(i, _):
        zero_v[pl.ds(i * 16, 16)] = zeros16
        return 0

    lax.fori_loop(0, 640 // 16, fillz, 0)

    pltpu.sync_copy(zero_v, deg_sh.at[pl.ds(sid * 640, 640)])
    plsc.subcore_barrier()

    def blk(bi, _):
        pltpu.sync_copy(col2d_hbm.at[pl.ds(sid * EROWS_PT + bi * JBLK, JBLK)],
                        idx_v)

        def sub(j, _):
            pltpu.sync_copy(ones_v, deg_sh.at[idx_v.at[j]], add=True)
            return 0

        lax.fori_loop(0, JBLK, sub, 0)
        return 0

    lax.fori_loop(0, NBLK, blk, 0)
    plsc.subcore_barrier()

    pltpu.sync_copy(deg_sh.at[pl.ds(sid * 640, 640)],
                    deg_hbm.at[pl.ds(sid * 640, 640)])


_deg_kernel = functools.partial(
    pl.kernel,
    out_type=jax.ShapeDtypeStruct((DEGPAD,), jnp.float32),
    mesh=plsc.VectorSubcoreMesh(core_axis_name="c", subcore_axis_name="s",
                                num_cores=1),
    scratch_types=[
        pltpu.VMEM_SHARED((DEGPAD,), jnp.float32),
        pltpu.VMEM((JBLK, 128), jnp.int32),
        pltpu.VMEM((128,), jnp.float32),
        pltpu.VMEM((640,), jnp.float32),
        pltpu.SemaphoreType.DMA,
    ],
)(_deg_body)


def _prologue_body(x_ref, w_ref, b_ref, deg_ref, p0_ref, c2_ref, q_ref,
                   sq_ref):
    x = x_ref[...]
    w = w_ref[...]
    h = lax.dot_general(x, w, (((1,), (1,)), ((), ())),
                        preferred_element_type=jnp.float32)
    h = jnp.maximum(h + b_ref[...], 0.0)
    deg = deg_ref[...] + 1.0
    dis = lax.rsqrt(deg)
    p0_ref[...] = dis * h
    c2_ref[...] = jnp.broadcast_to((1.0 - ALPHA) * dis * dis,
                                   h.shape).astype(jnp.float32)
    q_ref[...] = ALPHA * dis * h
    sq_ref[...] = jnp.broadcast_to(jnp.sqrt(deg), h.shape).astype(jnp.float32)


def _prologue(x, w, b2d, deg2d):
    blk = 1000
    grid = (N // blk,)
    out_shapes = [jax.ShapeDtypeStruct((N, D), jnp.float32)] * 4
    return pl.pallas_call(
        _prologue_body,
        grid=grid,
        in_specs=[
            pl.BlockSpec((blk, D), lambda i: (i, 0)),
            pl.BlockSpec((D, D), lambda i: (0, 0)),
            pl.BlockSpec((1, D), lambda i: (0, 0)),
            pl.BlockSpec((blk, 1), lambda i: (i, 0)),
        ],
        out_specs=[pl.BlockSpec((blk, D), lambda i: (i, 0))] * 4,
        out_shape=out_shapes,
    )(x, w, b2d, deg2d)


def _prop_body(p0_hbm, row2d_hbm, col2d_hbm, c2_hbm, q_hbm, p_hbm,
               s_sh, idxr_v, idxc_v, rows_v, sbuf, pbuf, qbuf, c2buf,
               zbuf, sem):
    sid = lax.axis_index("s")
    rbase = sid * ROWS_PT

    zeros16 = jnp.zeros((16,), jnp.float32)

    def fillz(i, _):
        r = i // 8
        c = (i % 8) * 16
        zbuf[r, pl.ds(c, 16)] = zeros16
        return 0

    lax.fori_loop(0, ZR * 8, fillz, 0)

    def zero_s(ro):
        def zc(zi, _):
            pltpu.sync_copy(zbuf, s_sh.at[pl.ds(ro + zi * ZR, ZR)])
            return 0

        lax.fori_loop(0, RCH // ZR, zc, 0)

    def init(ci, _):
        ro = rbase + ci * RCH
        pltpu.sync_copy(p0_hbm.at[pl.ds(ro, RCH)], sbuf)
        pltpu.sync_copy(sbuf, p_hbm.at[pl.ds(ro, RCH)])
        zero_s(ro)
        return 0

    lax.fori_loop(0, NCH, init, 0)
    plsc.subcore_barrier()

    def hop(_k, _):
        def blk(bi, _):
            ibase = sid * EROWS_PT + bi * JBLK
            pltpu.sync_copy(row2d_hbm.at[pl.ds(ibase, JBLK)], idxr_v)
            pltpu.sync_copy(col2d_hbm.at[pl.ds(ibase, JBLK)], idxc_v)

            def sub(j, _):
                pltpu.async_copy(p_hbm.at[idxr_v.at[j]], rows_v, sem).wait()
                pltpu.sync_copy(rows_v, s_sh.at[idxc_v.at[j]], add=True)
                return 0

            lax.fori_loop(0, JBLK, sub, 0)
            return 0

        lax.fori_loop(0, NBLK, blk, 0)
        plsc.subcore_barrier()

        def ew(ci, _):
            ro = rbase + ci * RCH
            pltpu.sync_copy(s_sh.at[pl.ds(ro, RCH)], sbuf)
            pltpu.sync_copy(p_hbm.at[pl.ds(ro, RCH)], pbuf)
            pltpu.sync_copy(c2_hbm.at[pl.ds(ro, RCH)], c2buf)
            pltpu.sync_copy(q_hbm.at[pl.ds(ro, RCH)], qbuf)

            def compute(i, _):
                r = i // 8
                c = (i % 8) * 16
                s = pl.ds(c, 16)
                v = c2buf[r, s] * (sbuf[r, s] + pbuf[r, s]) + qbuf[r, s]
                sbuf[r, s] = v
                return 0

            lax.fori_loop(0, RCH * 8, compute, 0)
            pltpu.sync_copy(sbuf, p_hbm.at[pl.ds(ro, RCH)])
            zero_s(ro)
            return 0

        lax.fori_loop(0, NCH, ew, 0)
        plsc.subcore_barrier()
        return 0

    lax.fori_loop(0, K, hop, 0)


_prop_kernel = functools.partial(
    pl.kernel,
    out_type=jax.ShapeDtypeStruct((NPAD, D), jnp.float32),
    mesh=plsc.VectorSubcoreMesh(core_axis_name="c", subcore_axis_name="s",
                                num_cores=1),
    scratch_types=[
        pltpu.VMEM_SHARED((NPAD, D), jnp.float32),
        pltpu.VMEM((JBLK, 128), jnp.int32),
        pltpu.VMEM((JBLK, 128), jnp.int32),
        pltpu.VMEM((128, D), jnp.float32),
        pltpu.VMEM((RCH, D), jnp.float32),
        pltpu.VMEM((RCH, D), jnp.float32),
        pltpu.VMEM((RCH, D), jnp.float32),
        pltpu.VMEM((RCH, D), jnp.float32),
        pltpu.VMEM((ZR, D), jnp.float32),
        pltpu.SemaphoreType.DMA,
    ],
)(_prop_body)


def _epilogue_body(p_ref, sq_ref, o_ref):
    o_ref[...] = p_ref[...] * sq_ref[...]


def _epilogue(p, sq):
    blk = 1000
    return pl.pallas_call(
        _epilogue_body,
        grid=(N // blk,),
        in_specs=[pl.BlockSpec((blk, D), lambda i: (i, 0))] * 2,
        out_specs=pl.BlockSpec((blk, D), lambda i: (i, 0)),
        out_shape=jax.ShapeDtypeStruct((N, D), jnp.float32),
    )(p, sq)


def kernel(x, edge_index, W, b):
    row = edge_index[0]
    col = edge_index[1]
    pad = EPAD - E
    row2d = jnp.concatenate(
        [row, jnp.zeros((pad,), jnp.int32)]).reshape(EROWS, 128)
    col2d = jnp.concatenate(
        [col, jnp.full((pad,), N, jnp.int32)]).reshape(EROWS, 128)

    deg = _deg_kernel(col2d)
    deg2d = deg[:N].reshape(N, 1)
    p0, c2, q, sq = _prologue(x, W, b.reshape(1, D), deg2d)
    zpad = jnp.zeros((NPAD - N, D), jnp.float32)
    p = _prop_kernel(jnp.concatenate([p0, zpad]),
                     row2d, col2d,
                     jnp.concatenate([c2, zpad]),
                     jnp.concatenate([q, zpad]))
    return _epilogue(p[:N], sq)


# speedup vs baseline: 5.3155x; 1.5031x over previous
import functools

import jax
import jax.numpy as jnp
from jax import lax
from jax.experimental import pallas as pl
from jax.experimental.pallas import tpu as pltpu
from jax.experimental.pallas import tpu_sc as plsc

N = 10000
E = 320000
D = 128
K = 10
ALPHA = 0.1

NC = 2
NS = 16
NW = NC * NS
U = 64
EPAD = 5120 * U
EROWS = EPAD // U
EROWS_PT = EROWS // NW
JBLK = 8
NRING = 4
UNITS_PT = EROWS_PT
NITER = UNITS_PT // NRING
NPAD = 10240
ROWS_PT = NPAD // NS
ZR = 64
DEGPAD = NPAD


DEG_EROWS = EPAD // 128
DEG_EROWS_PT = DEG_EROWS // NS


def _deg_body(col2d_hbm, deg_hbm, deg_sh, idx_v, ones_v, zero_v, sem):
    del sem
    sid = lax.axis_index("s")

    ones16 = jnp.ones((16,), jnp.float32)
    zeros16 = jnp.zeros((16,), jnp.float32)

    def fill(i, _):
        ones_v[pl.ds(i * 16, 16)] = ones16
        return 0

    lax.fori_loop(0, 128 // 16, fill, 0)

    def fillz(i, _):
        zero_v[pl.ds(i * 16, 16)] = zeros16
        return 0

    lax.fori_loop(0, 640 // 16, fillz, 0)

    pltpu.sync_copy(zero_v, deg_sh.at[pl.ds(sid * 640, 640)])
    plsc.subcore_barrier()

    def blk(bi, _):
        pltpu.sync_copy(
            col2d_hbm.at[pl.ds(sid * DEG_EROWS_PT + bi * JBLK, JBLK)], idx_v)

        def sub(j, _):
            pltpu.sync_copy(ones_v, deg_sh.at[idx_v.at[j]], add=True)
            return 0

        lax.fori_loop(0, JBLK, sub, 0)
        return 0

    lax.fori_loop(0, DEG_EROWS_PT // JBLK, blk, 0)
    plsc.subcore_barrier()

    pltpu.sync_copy(deg_sh.at[pl.ds(sid * 640, 640)],
                    deg_hbm.at[pl.ds(sid * 640, 640)])


_deg_kernel = functools.partial(
    pl.kernel,
    out_type=jax.ShapeDtypeStruct((DEGPAD,), jnp.float32),
    mesh=plsc.VectorSubcoreMesh(core_axis_name="c", subcore_axis_name="s",
                                num_cores=1),
    scratch_types=[
        pltpu.VMEM_SHARED((DEGPAD,), jnp.float32),
        pltpu.VMEM((JBLK, 128), jnp.int32),
        pltpu.VMEM((128,), jnp.float32),
        pltpu.VMEM((640,), jnp.float32),
        pltpu.SemaphoreType.DMA,
    ],
)(_deg_body)


def _prologue_body(x_ref, w_ref, b_ref, deg_ref, p0_ref, c2_ref, q_ref,
                   sq_ref):
    x = x_ref[...]
    w = w_ref[...]
    h = lax.dot_general(x, w, (((1,), (1,)), ((), ())),
                        preferred_element_type=jnp.float32)
    h = jnp.maximum(h + b_ref[...], 0.0)
    deg = deg_ref[...] + 1.0
    dis = lax.rsqrt(deg)
    p0_ref[...] = dis * h
    c2_ref[...] = jnp.broadcast_to((1.0 - ALPHA) * dis * dis,
                                   h.shape).astype(jnp.float32)
    q_ref[...] = ALPHA * dis * h
    sq_ref[...] = jnp.broadcast_to(jnp.sqrt(deg), h.shape).astype(jnp.float32)


def _prologue(x, w, b2d, deg2d):
    blk = 1000
    out_shapes = [jax.ShapeDtypeStruct((N, D), jnp.float32)] * 4
    return pl.pallas_call(
        _prologue_body,
        grid=(N // blk,),
        in_specs=[
            pl.BlockSpec((blk, D), lambda i: (i, 0)),
            pl.BlockSpec((D, D), lambda i: (0, 0)),
            pl.BlockSpec((1, D), lambda i: (0, 0)),
            pl.BlockSpec((blk, 1), lambda i: (i, 0)),
        ],
        out_specs=[pl.BlockSpec((blk, D), lambda i: (i, 0))] * 4,
        out_shape=out_shapes,
    )(x, w, b2d, deg2d)


def _hop_body(p_hbm, row2d_hbm, col2d_hbm, s_out,
              s_sh, idxr, idxc, r0, r1, r2, r3, zbuf,
              g0, g1, g2, g3, s0, s1, s2, s3):
    cid = lax.axis_index("c")
    sid = lax.axis_index("s")
    rows = (r0, r1, r2, r3)
    gsem = (g0, g1, g2, g3)
    ssem = (s0, s1, s2, s3)
    base_r = (cid * NS + sid) * EROWS_PT

    zeros16 = jnp.zeros((16,), jnp.float32)

    def fillz(i, _):
        zbuf[i // 8, pl.ds((i % 8) * 16, 16)] = zeros16
        return 0

    lax.fori_loop(0, ZR * 8, fillz, 0)

    def zc(zi, _):
        pltpu.sync_copy(zbuf, s_sh.at[pl.ds(sid * ROWS_PT + zi * ZR, ZR)])
        return 0

    lax.fori_loop(0, ROWS_PT // ZR, zc, 0)
    plsc.subcore_barrier()

    def it(i, _):
        par = (i // 2) % 2

        @pl.when(i > 0)
        def _drain():
            for u in range(NRING):
                pltpu.make_async_copy(
                    rows[u], s_sh.at[idxc.at[0, u]], ssem[u]).wait()

        @pl.when(i % 2 == 0)
        def _loadidx():
            b = i // 2
            pltpu.sync_copy(row2d_hbm.at[pl.ds(base_r + b * JBLK, JBLK)],
                            idxr.at[par])
            pltpu.sync_copy(col2d_hbm.at[pl.ds(base_r + b * JBLK, JBLK)],
                            idxc.at[par])

        j0 = (i % 2) * NRING
        gd = []
        for u in range(NRING):
            gd.append(pltpu.async_copy(p_hbm.at[idxr.at[par, j0 + u]],
                                       rows[u], gsem[u]))
        for u in range(NRING):
            gd[u].wait()
            pltpu.async_copy(rows[u], s_sh.at[idxc.at[par, j0 + u]],
                             ssem[u], add=True)
        return 0

    lax.fori_loop(0, NITER, it, 0)
    for u in range(NRING):
        pltpu.make_async_copy(rows[u], s_sh.at[idxc.at[0, u]],
                              ssem[u]).wait()
    plsc.subcore_barrier()

    pltpu.sync_copy(s_sh.at[pl.ds(sid * ROWS_PT, ROWS_PT)],
                    s_out.at[cid, pl.ds(sid * ROWS_PT, ROWS_PT)])


_hop_kernel = functools.partial(
    pl.kernel,
    out_type=jax.ShapeDtypeStruct((NC, NPAD, D), jnp.float32),
    mesh=plsc.VectorSubcoreMesh(core_axis_name="c", subcore_axis_name="s"),
    scratch_types=[
        pltpu.VMEM_SHARED((NPAD, D), jnp.float32),
        pltpu.VMEM((2, JBLK, U), jnp.int32),
        pltpu.VMEM((2, JBLK, U), jnp.int32),
        pltpu.VMEM((U, D), jnp.float32),
        pltpu.VMEM((U, D), jnp.float32),
        pltpu.VMEM((U, D), jnp.float32),
        pltpu.VMEM((U, D), jnp.float32),
        pltpu.VMEM((ZR, D), jnp.float32),
        pltpu.SemaphoreType.DMA,
        pltpu.SemaphoreType.DMA,
        pltpu.SemaphoreType.DMA,
        pltpu.SemaphoreType.DMA,
        pltpu.SemaphoreType.DMA,
        pltpu.SemaphoreType.DMA,
        pltpu.SemaphoreType.DMA,
        pltpu.SemaphoreType.DMA,
    ],
)(_hop_body)


def _ew_body(s_ref, p_ref, c2_ref, q_ref, o_ref):
    s = s_ref[0] + s_ref[1]
    o_ref[...] = c2_ref[...] * (s + p_ref[...]) + q_ref[...]


def _ew_final_body(s_ref, p_ref, c2_ref, q_ref, sq_ref, o_ref):
    s = s_ref[0] + s_ref[1]
    o_ref[...] = (c2_ref[...] * (s + p_ref[...]) + q_ref[...]) * sq_ref[...]


def _ew(s2, p, c2, q, sq=None):
    blk = 1024
    nio = 4 if sq is None else 5
    body = _ew_body if sq is None else _ew_final_body
    args = (s2, p, c2, q) if sq is None else (s2, p, c2, q, sq)
    in_specs = [pl.BlockSpec((NC, blk, D), lambda i: (0, i, 0))]
    in_specs += [pl.BlockSpec((blk, D), lambda i: (i, 0))] * (nio - 1)
    return pl.pallas_call(
        body,
        grid=(NPAD // blk,),
        in_specs=in_specs,
        out_specs=pl.BlockSpec((blk, D), lambda i: (i, 0)),
        out_shape=jax.ShapeDtypeStruct((NPAD, D), jnp.float32),
    )(*args)


def kernel(x, edge_index, W, b):
    row = edge_index[0]
    col = edge_index[1]
    pad = EPAD - E
    row2d = jnp.concatenate(
        [row, jnp.zeros((pad,), jnp.int32)]).reshape(EROWS, U)
    col2d = jnp.concatenate(
        [col, jnp.full((pad,), N, jnp.int32)]).reshape(EROWS, U)
    col128 = col2d.reshape(DEG_EROWS, 128)

    deg = _deg_kernel(col128)
    deg2d = deg[:N].reshape(N, 1)
    p0, c2, q, sq = _prologue(x, W, b.reshape(1, D), deg2d)
    zpadrows = jnp.zeros((NPAD - N, D), jnp.float32)
    p = jnp.concatenate([p0, zpadrows])
    c2p = jnp.concatenate([c2, zpadrows])
    qp = jnp.concatenate([q, zpadrows])
    sqp = jnp.concatenate([sq, zpadrows])

    for k in range(K):
        s2 = _hop_kernel(p, row2d, col2d)
        if k < K - 1:
            p = _ew(s2, p, c2p, qp)
        else:
            p = _ew(s2, p, c2p, qp, sqp)
    return p[:N]


# speedup vs baseline: 5.3312x; 1.0030x over previous
import functools

import jax
import jax.numpy as jnp
from jax import lax
from jax.experimental import pallas as pl
from jax.experimental.pallas import tpu as pltpu
from jax.experimental.pallas import tpu_sc as plsc

N = 10000
E = 320000
D = 128
K = 10
ALPHA = 0.1

NC = 2
NS = 16
NW = NC * NS
U = 128
EPAD = 2560 * U
EROWS = EPAD // U
EROWS_PT = EROWS // NW
JBLK = 8
NRING = 2
QG = 4
UQ = U // QG
UNITS_PT = EROWS_PT
NITER = UNITS_PT // NRING
NPAD = 10240
ROWS_PT = NPAD // NS
ZR = 64
DEGPAD = NPAD


DEG_EROWS = EPAD // 128
DEG_EROWS_PT = DEG_EROWS // NS


def _deg_body(col2d_hbm, deg_hbm, deg_sh, idx_v, ones_v, zero_v, sem):
    del sem
    sid = lax.axis_index("s")

    ones16 = jnp.ones((16,), jnp.float32)
    zeros16 = jnp.zeros((16,), jnp.float32)

    def fill(i, _):
        ones_v[pl.ds(i * 16, 16)] = ones16
        return 0

    lax.fori_loop(0, 128 // 16, fill, 0)

    def fillz(i, _):
        zero_v[pl.ds(i * 16, 16)] = zeros16
        return 0

    lax.fori_loop(0, 640 // 16, fillz, 0)

    pltpu.sync_copy(zero_v, deg_sh.at[pl.ds(sid * 640, 640)])
    plsc.subcore_barrier()

    def blk(bi, _):
        pltpu.sync_copy(
            col2d_hbm.at[pl.ds(sid * DEG_EROWS_PT + bi * JBLK, JBLK)], idx_v)

        def sub(j, _):
            pltpu.sync_copy(ones_v, deg_sh.at[idx_v.at[j]], add=True)
            return 0

        lax.fori_loop(0, JBLK, sub, 0)
        return 0

    lax.fori_loop(0, DEG_EROWS_PT // JBLK, blk, 0)
    plsc.subcore_barrier()

    pltpu.sync_copy(deg_sh.at[pl.ds(sid * 640, 640)],
                    deg_hbm.at[pl.ds(sid * 640, 640)])


_deg_kernel = functools.partial(
    pl.kernel,
    out_type=jax.ShapeDtypeStruct((DEGPAD,), jnp.float32),
    mesh=plsc.VectorSubcoreMesh(core_axis_name="c", subcore_axis_name="s",
                                num_cores=1),
    scratch_types=[
        pltpu.VMEM_SHARED((DEGPAD,), jnp.float32),
        pltpu.VMEM((JBLK, 128), jnp.int32),
        pltpu.VMEM((128,), jnp.float32),
        pltpu.VMEM((640,), jnp.float32),
        pltpu.SemaphoreType.DMA,
    ],
)(_deg_body)


def _prologue_body(x_ref, w_ref, b_ref, deg_ref, p0_ref, c2_ref, q_ref,
                   sq_ref):
    x = x_ref[...]
    w = w_ref[...]
    h = lax.dot_general(x, w, (((1,), (1,)), ((), ())),
                        preferred_element_type=jnp.float32)
    h = jnp.maximum(h + b_ref[...], 0.0)
    deg = deg_ref[...] + 1.0
    dis = lax.rsqrt(deg)
    p0_ref[...] = dis * h
    c2_ref[...] = jnp.broadcast_to((1.0 - ALPHA) * dis * dis,
                                   h.shape).astype(jnp.float32)
    q_ref[...] = ALPHA * dis * h
    sq_ref[...] = jnp.broadcast_to(jnp.sqrt(deg), h.shape).astype(jnp.float32)


def _prologue(x, w, b2d, deg2d):
    blk = 1000
    out_shapes = [jax.ShapeDtypeStruct((N, D), jnp.float32)] * 4
    return pl.pallas_call(
        _prologue_body,
        grid=(N // blk,),
        in_specs=[
            pl.BlockSpec((blk, D), lambda i: (i, 0)),
            pl.BlockSpec((D, D), lambda i: (0, 0)),
            pl.BlockSpec((1, D), lambda i: (0, 0)),
            pl.BlockSpec((blk, 1), lambda i: (i, 0)),
        ],
        out_specs=[pl.BlockSpec((blk, D), lambda i: (i, 0))] * 4,
        out_shape=out_shapes,
    )(x, w, b2d, deg2d)


def _hop_body(p_hbm, row2d_hbm, col2d_hbm, s_out,
              s_sh, idxr, idxc, r0, r1, zbuf,
              g00, g01, g02, g03, g10, g11, g12, g13, s0, s1):
    cid = lax.axis_index("c")
    sid = lax.axis_index("s")
    rows = (r0, r1)
    gsem = ((g00, g01, g02, g03), (g10, g11, g12, g13))
    ssem = (s0, s1)
    base_r = (cid * NS + sid) * EROWS_PT

    zeros16 = jnp.zeros((16,), jnp.float32)

    def fillz(i, _):
        zbuf[i // 8, pl.ds((i % 8) * 16, 16)] = zeros16
        return 0

    lax.fori_loop(0, ZR * 8, fillz, 0)

    def zc(zi, _):
        pltpu.sync_copy(zbuf, s_sh.at[pl.ds(sid * ROWS_PT + zi * ZR, ZR)])
        return 0

    lax.fori_loop(0, ROWS_PT // ZR, zc, 0)
    plsc.subcore_barrier()

    def it(i, _):
        par = (i // 4) % 2

        @pl.when(i > 0)
        def _drain():
            for u in range(NRING):
                pltpu.make_async_copy(
                    rows[u], s_sh.at[idxc.at[0, u]], ssem[u]).wait()

        @pl.when(i % 4 == 0)
        def _loadidx():
            b = i // 4
            pltpu.sync_copy(row2d_hbm.at[pl.ds(base_r + b * JBLK, JBLK)],
                            idxr.at[par])
            pltpu.sync_copy(col2d_hbm.at[pl.ds(base_r + b * JBLK, JBLK)],
                            idxc.at[par])

        j0 = (i % 4) * NRING
        gd = []
        for u in range(NRING):
            for qg in range(QG):
                gd.append(pltpu.async_copy(
                    p_hbm.at[idxr.at[par, j0 + u, pl.ds(qg * UQ, UQ)]],
                    rows[u].at[pl.ds(qg * UQ, UQ)], gsem[u][qg]))
        for u in range(NRING):
            for qg in range(QG):
                gd[u * QG + qg].wait()
            pltpu.async_copy(rows[u], s_sh.at[idxc.at[par, j0 + u]],
                             ssem[u], add=True)
        return 0

    lax.fori_loop(0, NITER, it, 0)
    for u in range(NRING):
        pltpu.make_async_copy(rows[u], s_sh.at[idxc.at[0, u]],
                              ssem[u]).wait()
    plsc.subcore_barrier()

    pltpu.sync_copy(s_sh.at[pl.ds(sid * ROWS_PT, ROWS_PT)],
                    s_out.at[cid, pl.ds(sid * ROWS_PT, ROWS_PT)])


_hop_kernel = functools.partial(
    pl.kernel,
    out_type=jax.ShapeDtypeStruct((NC, NPAD, D), jnp.float32),
    mesh=plsc.VectorSubcoreMesh(core_axis_name="c", subcore_axis_name="s"),
    scratch_types=[
        pltpu.VMEM_SHARED((NPAD, D), jnp.float32),
        pltpu.VMEM((2, JBLK, U), jnp.int32),
        pltpu.VMEM((2, JBLK, U), jnp.int32),
        pltpu.VMEM((U, D), jnp.float32),
        pltpu.VMEM((U, D), jnp.float32),
        pltpu.VMEM((ZR, D), jnp.float32),
    ] + [pltpu.SemaphoreType.DMA] * 10,
)(_hop_body)


def _ew_body(s_ref, p_ref, c2_ref, q_ref, o_ref):
    s = s_ref[0] + s_ref[1]
    o_ref[...] = c2_ref[...] * (s + p_ref[...]) + q_ref[...]


def _ew_final_body(s_ref, p_ref, c2_ref, q_ref, sq_ref, o_ref):
    s = s_ref[0] + s_ref[1]
    o_ref[...] = (c2_ref[...] * (s + p_ref[...]) + q_ref[...]) * sq_ref[...]


def _ew(s2, p, c2, q, sq=None):
    blk = 1024
    nio = 4 if sq is None else 5
    body = _ew_body if sq is None else _ew_final_body
    args = (s2, p, c2, q) if sq is None else (s2, p, c2, q, sq)
    in_specs = [pl.BlockSpec((NC, blk, D), lambda i: (0, i, 0))]
    in_specs += [pl.BlockSpec((blk, D), lambda i: (i, 0))] * (nio - 1)
    return pl.pallas_call(
        body,
        grid=(NPAD // blk,),
        in_specs=in_specs,
        out_specs=pl.BlockSpec((blk, D), lambda i: (i, 0)),
        out_shape=jax.ShapeDtypeStruct((NPAD, D), jnp.float32),
    )(*args)


def kernel(x, edge_index, W, b):
    row = edge_index[0]
    col = edge_index[1]
    pad = EPAD - E
    row2d = jnp.concatenate(
        [row, jnp.zeros((pad,), jnp.int32)]).reshape(EROWS, U)
    col2d = jnp.concatenate(
        [col, jnp.full((pad,), N, jnp.int32)]).reshape(EROWS, U)
    col128 = col2d.reshape(DEG_EROWS, 128)

    deg = _deg_kernel(col128)
    deg2d = deg[:N].reshape(N, 1)
    p0, c2, q, sq = _prologue(x, W, b.reshape(1, D), deg2d)
    zpadrows = jnp.zeros((NPAD - N, D), jnp.float32)
    p = jnp.concatenate([p0, zpadrows])
    c2p = jnp.concatenate([c2, zpadrows])
    qp = jnp.concatenate([q, zpadrows])
    sqp = jnp.concatenate([sq, zpadrows])

    for k in range(K):
        s2 = _hop_kernel(p, row2d, col2d)
        if k < K - 1:
            p = _ew(s2, p, c2p, qp)
        else:
            p = _ew(s2, p, c2p, qp, sqp)
    return p[:N]


# speedup vs baseline: 5.3432x; 1.0022x over previous
import functools

import jax
import jax.numpy as jnp
from jax import lax
from jax.experimental import pallas as pl
from jax.experimental.pallas import tpu as pltpu
from jax.experimental.pallas import tpu_sc as plsc

N = 10000
E = 320000
D = 128
K = 10
ALPHA = 0.1

NC = 2
NS = 16
NW = NC * NS
U = 128
EPAD = 2560 * U
EROWS = EPAD // U
EROWS_PT = EROWS // NW
JBLK = 8
NRING = 2
QG = 8
UQ = U // QG
UNITS_PT = EROWS_PT
NITER = UNITS_PT // NRING
NPAD = 10240
ROWS_PT = NPAD // NS
ZR = 64
DEGPAD = NPAD


DEG_EROWS = EPAD // 128
DEG_EROWS_PT = DEG_EROWS // NS


def _deg_body(col2d_hbm, deg_hbm, deg_sh, idx_v, ones_v, zero_v, sem):
    del sem
    sid = lax.axis_index("s")

    ones16 = jnp.ones((16,), jnp.float32)
    zeros16 = jnp.zeros((16,), jnp.float32)

    def fill(i, _):
        ones_v[pl.ds(i * 16, 16)] = ones16
        return 0

    lax.fori_loop(0, 128 // 16, fill, 0)

    def fillz(i, _):
        zero_v[pl.ds(i * 16, 16)] = zeros16
        return 0

    lax.fori_loop(0, 640 // 16, fillz, 0)

    pltpu.sync_copy(zero_v, deg_sh.at[pl.ds(sid * 640, 640)])
    plsc.subcore_barrier()

    def blk(bi, _):
        pltpu.sync_copy(
            col2d_hbm.at[pl.ds(sid * DEG_EROWS_PT + bi * JBLK, JBLK)], idx_v)

        def sub(j, _):
            pltpu.sync_copy(ones_v, deg_sh.at[idx_v.at[j]], add=True)
            return 0

        lax.fori_loop(0, JBLK, sub, 0)
        return 0

    lax.fori_loop(0, DEG_EROWS_PT // JBLK, blk, 0)
    plsc.subcore_barrier()

    pltpu.sync_copy(deg_sh.at[pl.ds(sid * 640, 640)],
                    deg_hbm.at[pl.ds(sid * 640, 640)])


_deg_kernel = functools.partial(
    pl.kernel,
    out_type=jax.ShapeDtypeStruct((DEGPAD,), jnp.float32),
    mesh=plsc.VectorSubcoreMesh(core_axis_name="c", subcore_axis_name="s",
                                num_cores=1),
    scratch_types=[
        pltpu.VMEM_SHARED((DEGPAD,), jnp.float32),
        pltpu.VMEM((JBLK, 128), jnp.int32),
        pltpu.VMEM((128,), jnp.float32),
        pltpu.VMEM((640,), jnp.float32),
        pltpu.SemaphoreType.DMA,
    ],
)(_deg_body)


def _prologue_body(x_ref, w_ref, b_ref, deg_ref, p0_ref, c2_ref, q_ref,
                   sq_ref):
    x = x_ref[...]
    w = w_ref[...]
    h = lax.dot_general(x, w, (((1,), (1,)), ((), ())),
                        preferred_element_type=jnp.float32)
    h = jnp.maximum(h + b_ref[...], 0.0)
    deg = deg_ref[...] + 1.0
    dis = lax.rsqrt(deg)
    p0_ref[...] = dis * h
    c2_ref[...] = jnp.broadcast_to((1.0 - ALPHA) * dis * dis,
                                   h.shape).astype(jnp.float32)
    q_ref[...] = ALPHA * dis * h
    sq_ref[...] = jnp.broadcast_to(jnp.sqrt(deg), h.shape).astype(jnp.float32)


def _prologue(x, w, b2d, deg2d):
    blk = 1000
    out_shapes = [jax.ShapeDtypeStruct((N, D), jnp.float32)] * 4
    return pl.pallas_call(
        _prologue_body,
        grid=(N // blk,),
        in_specs=[
            pl.BlockSpec((blk, D), lambda i: (i, 0)),
            pl.BlockSpec((D, D), lambda i: (0, 0)),
            pl.BlockSpec((1, D), lambda i: (0, 0)),
            pl.BlockSpec((blk, 1), lambda i: (i, 0)),
        ],
        out_specs=[pl.BlockSpec((blk, D), lambda i: (i, 0))] * 4,
        out_shape=out_shapes,
    )(x, w, b2d, deg2d)


def _hop_body(p_hbm, row2d_hbm, col2d_hbm, s_out,
              s_sh, idxr, idxc, r0, r1, zbuf,
              g00, g01, g02, g03, g04, g05, g06, g07,
              g10, g11, g12, g13, g14, g15, g16, g17, s0, s1):
    cid = lax.axis_index("c")
    sid = lax.axis_index("s")
    rows = (r0, r1)
    gsem = ((g00, g01, g02, g03, g04, g05, g06, g07),
            (g10, g11, g12, g13, g14, g15, g16, g17))
    ssem = (s0, s1)
    base_r = (cid * NS + sid) * EROWS_PT

    zeros16 = jnp.zeros((16,), jnp.float32)

    def fillz(i, _):
        zbuf[i // 8, pl.ds((i % 8) * 16, 16)] = zeros16
        return 0

    lax.fori_loop(0, ZR * 8, fillz, 0)

    def zc(zi, _):
        pltpu.sync_copy(zbuf, s_sh.at[pl.ds(sid * ROWS_PT + zi * ZR, ZR)])
        return 0

    lax.fori_loop(0, ROWS_PT // ZR, zc, 0)
    plsc.subcore_barrier()

    def it(i, _):
        par = (i // 4) % 2

        @pl.when(i > 0)
        def _drain():
            for u in range(NRING):
                pltpu.make_async_copy(
                    rows[u], s_sh.at[idxc.at[0, u]], ssem[u]).wait()

        @pl.when(i % 4 == 0)
        def _loadidx():
            b = i // 4
            pltpu.sync_copy(row2d_hbm.at[pl.ds(base_r + b * JBLK, JBLK)],
                            idxr.at[par])
            pltpu.sync_copy(col2d_hbm.at[pl.ds(base_r + b * JBLK, JBLK)],
                            idxc.at[par])

        j0 = (i % 4) * NRING
        gd = []
        for u in range(NRING):
            for qg in range(QG):
                gd.append(pltpu.async_copy(
                    p_hbm.at[idxr.at[par, j0 + u, pl.ds(qg * UQ, UQ)]],
                    rows[u].at[pl.ds(qg * UQ, UQ)], gsem[u][qg]))
        for u in range(NRING):
            for qg in range(QG):
                gd[u * QG + qg].wait()
            pltpu.async_copy(rows[u], s_sh.at[idxc.at[par, j0 + u]],
                             ssem[u], add=True)
        return 0

    lax.fori_loop(0, NITER, it, 0)
    for u in range(NRING):
        pltpu.make_async_copy(rows[u], s_sh.at[idxc.at[0, u]],
                              ssem[u]).wait()
    plsc.subcore_barrier()

    pltpu.sync_copy(s_sh.at[pl.ds(sid * ROWS_PT, ROWS_PT)],
                    s_out.at[cid, pl.ds(sid * ROWS_PT, ROWS_PT)])


_hop_kernel = functools.partial(
    pl.kernel,
    out_type=jax.ShapeDtypeStruct((NC, NPAD, D), jnp.float32),
    mesh=plsc.VectorSubcoreMesh(core_axis_name="c", subcore_axis_name="s"),
    scratch_types=[
        pltpu.VMEM_SHARED((NPAD, D), jnp.float32),
        pltpu.VMEM((2, JBLK, U), jnp.int32),
        pltpu.VMEM((2, JBLK, U), jnp.int32),
        pltpu.VMEM((U, D), jnp.float32),
        pltpu.VMEM((U, D), jnp.float32),
        pltpu.VMEM((ZR, D), jnp.float32),
    ] + [pltpu.SemaphoreType.DMA] * 18,
)(_hop_body)


def _ew_body(s_ref, p_ref, c2_ref, q_ref, o_ref):
    s = s_ref[0] + s_ref[1]
    o_ref[...] = c2_ref[...] * (s + p_ref[...]) + q_ref[...]


def _ew_final_body(s_ref, p_ref, c2_ref, q_ref, sq_ref, o_ref):
    s = s_ref[0] + s_ref[1]
    o_ref[...] = (c2_ref[...] * (s + p_ref[...]) + q_ref[...]) * sq_ref[...]


def _ew(s2, p, c2, q, sq=None):
    blk = 1024
    nio = 4 if sq is None else 5
    body = _ew_body if sq is None else _ew_final_body
    args = (s2, p, c2, q) if sq is None else (s2, p, c2, q, sq)
    in_specs = [pl.BlockSpec((NC, blk, D), lambda i: (0, i, 0))]
    in_specs += [pl.BlockSpec((blk, D), lambda i: (i, 0))] * (nio - 1)
    return pl.pallas_call(
        body,
        grid=(NPAD // blk,),
        in_specs=in_specs,
        out_specs=pl.BlockSpec((blk, D), lambda i: (i, 0)),
        out_shape=jax.ShapeDtypeStruct((NPAD, D), jnp.float32),
    )(*args)


def kernel(x, edge_index, W, b):
    row = edge_index[0]
    col = edge_index[1]
    pad = EPAD - E
    row2d = jnp.concatenate(
        [row, jnp.zeros((pad,), jnp.int32)]).reshape(EROWS, U)
    col2d = jnp.concatenate(
        [col, jnp.full((pad,), N, jnp.int32)]).reshape(EROWS, U)
    col128 = col2d.reshape(DEG_EROWS, 128)

    deg = _deg_kernel(col128)
    deg2d = deg[:N].reshape(N, 1)
    p0, c2, q, sq = _prologue(x, W, b.reshape(1, D), deg2d)
    zpadrows = jnp.zeros((NPAD - N, D), jnp.float32)
    p = jnp.concatenate([p0, zpadrows])
    c2p = jnp.concatenate([c2, zpadrows])
    qp = jnp.concatenate([q, zpadrows])
    sqp = jnp.concatenate([sq, zpadrows])

    for k in range(K):
        s2 = _hop_kernel(p, row2d, col2d)
        if k < K - 1:
            p = _ew(s2, p, c2p, qp)
        else:
            p = _ew(s2, p, c2p, qp, sqp)
    return p[:N]
